# Initial kernel scaffold; baseline (speedup 1.0000x reference)
import jax
import jax.numpy as jnp
from jax.experimental import pallas as pl


def kernel(x, edge_index, W1, b1, W2, b2):
    raise NotImplementedError("write your pallas kernel here")


# speedup vs baseline: 12.3384x; 12.3384x over previous
import functools

import jax
import jax.numpy as jnp
from jax import lax
from jax.experimental import pallas as pl
from jax.experimental.pallas import tpu as pltpu
from jax.experimental.pallas import tpu_sc as plsc

N = 10000
E = 160000
F_IN, H, C = 256, 512, 128

NT = 16
NB = 80
BW = 128
EPT = NB * BW
E_PAD = NT * EPT

ACC_R = 10240
RPT_Z = 640
RPT_W = 624
DEG_R = 10240

def _deg_body(dst_hbm, out_hbm, dstv, ones_row, ones_init, deg_sh):
    c = lax.axis_index("c")
    s = lax.axis_index("s")

    @pl.when(c == 0)
    def _():
        one16 = jnp.full((16,), 1.0, jnp.float32)

        def fill_row(i, carry):
            ones_row[pl.ds(i * 16, 16)] = one16
            return carry

        lax.fori_loop(0, BW // 16, fill_row, 0)

        def fill_init(i, carry):
            ones_init[pl.ds(i * 16, 16)] = one16
            return carry

        lax.fori_loop(0, 640 // 16, fill_init, 0)

        pltpu.sync_copy(ones_init, deg_sh.at[pl.ds(s * 640, 640)])
        pltpu.sync_copy(dst_hbm.at[s], dstv)
        plsc.subcore_barrier()

        def body(j, carry):
            pltpu.sync_copy(ones_row, deg_sh.at[dstv.at[j]], add=True)
            return carry

        lax.fori_loop(0, NB, body, 0)
        plsc.subcore_barrier()
        pltpu.sync_copy(deg_sh.at[pl.ds(s * 640, 640)],
                        out_hbm.at[pl.ds(s * 640, 640)])


@functools.cache
def _deg_kernel_build():
    return pl.kernel(
        _deg_body,
        out_type=jax.ShapeDtypeStruct((DEG_R,), jnp.float32),
        mesh=plsc.VectorSubcoreMesh(core_axis_name="c", subcore_axis_name="s"),
        scratch_types=[
            pltpu.VMEM((NB, BW), jnp.int32),
            pltpu.VMEM((BW,), jnp.float32),
            pltpu.VMEM((640,), jnp.float32),
            pltpu.VMEM_SHARED((DEG_R,), jnp.float32),
        ],
    )


def _deg_kernel(dst_t):
    return _deg_kernel_build()(dst_t)


def _make_agg(n_chunks, w):
    cpc = n_chunks // 2

    def body(h_hbm, srcoff_hbm, dst_hbm, out_hbm, srcv, dstv, buf, zbuf, acc_sh):
        c = lax.axis_index("c")
        s = lax.axis_index("s")
        z16 = jnp.zeros((16,), jnp.float32)

        def zrow(i, carry):
            for k in range(w // 16):
                zbuf[i, pl.ds(k * 16, 16)] = z16
            return carry

        lax.fori_loop(0, 64, zrow, 0)
        pltpu.sync_copy(dst_hbm.at[s], dstv)

        for chunk in range(cpc):
            g = c * cpc + chunk
            pltpu.sync_copy(srcoff_hbm.at[g, s], srcv)
            base = s * RPT_Z
            for m in range(RPT_Z // 64):
                pltpu.sync_copy(zbuf, acc_sh.at[pl.ds(base + m * 64, 64)])
            plsc.subcore_barrier()

            def ebody(j, carry):
                pltpu.sync_copy(h_hbm.at[srcv.at[j]], buf)
                pltpu.sync_copy(buf, acc_sh.at[dstv.at[j]], add=True)
                return carry

            lax.fori_loop(0, NB, ebody, 0)
            plsc.subcore_barrier()
            wb = s * RPT_W

            @pl.when(s < NT - 1)
            def _():
                pltpu.sync_copy(acc_sh.at[pl.ds(wb, RPT_W)],
                                out_hbm.at[g, pl.ds(wb, RPT_W)])

            @pl.when(s == NT - 1)
            def _():
                pltpu.sync_copy(acc_sh.at[pl.ds((NT - 1) * RPT_W, N - (NT - 1) * RPT_W)],
                                out_hbm.at[g, pl.ds((NT - 1) * RPT_W, N - (NT - 1) * RPT_W)])

            plsc.subcore_barrier()

    return pl.kernel(
        body,
        out_type=jax.ShapeDtypeStruct((n_chunks, N, w), jnp.float32),
        mesh=plsc.VectorSubcoreMesh(core_axis_name="c", subcore_axis_name="s"),
        scratch_types=[
            pltpu.VMEM((NB, BW), jnp.int32),
            pltpu.VMEM((NB, BW), jnp.int32),
            pltpu.VMEM((BW, w), jnp.float32),
            pltpu.VMEM((64, w), jnp.float32),
            pltpu.VMEM_SHARED((ACC_R, w), jnp.float32),
        ],
    )


_make_agg = functools.cache(_make_agg)


def _agg4(hflat, src_c, dst_t):
    return _make_agg(4, 128)(hflat, src_c, dst_t)


NB2 = NB // 2


def _agg2_body(h_hbm, src_hbm, dst_hbm, out_hbm, srcv, dstv, buf, zbuf, acc_sh):
    c = lax.axis_index("c")
    s = lax.axis_index("s")
    z16 = jnp.zeros((16,), jnp.float32)

    def zrow(i, carry):
        for k in range(C // 16):
            zbuf[i, pl.ds(k * 16, 16)] = z16
        return carry

    lax.fori_loop(0, 64, zrow, 0)
    pltpu.sync_copy(dst_hbm.at[c, s], dstv)
    pltpu.sync_copy(src_hbm.at[c, s], srcv)
    base = s * RPT_Z
    for m in range(RPT_Z // 64):
        pltpu.sync_copy(zbuf, acc_sh.at[pl.ds(base + m * 64, 64)])
    plsc.subcore_barrier()

    def ebody(j, carry):
        pltpu.sync_copy(h_hbm.at[srcv.at[j]], buf)
        pltpu.sync_copy(buf, acc_sh.at[dstv.at[j]], add=True)
        return carry

    lax.fori_loop(0, NB2, ebody, 0)
    plsc.subcore_barrier()
    wb = s * RPT_W

    @pl.when(s < NT - 1)
    def _():
        pltpu.sync_copy(acc_sh.at[pl.ds(wb, RPT_W)],
                        out_hbm.at[c, pl.ds(wb, RPT_W)])

    @pl.when(s == NT - 1)
    def _():
        pltpu.sync_copy(acc_sh.at[pl.ds((NT - 1) * RPT_W, N - (NT - 1) * RPT_W)],
                        out_hbm.at[c, pl.ds((NT - 1) * RPT_W, N - (NT - 1) * RPT_W)])


@functools.cache
def _agg2_build():
    return pl.kernel(
        _agg2_body,
        out_type=jax.ShapeDtypeStruct((2, N, C), jnp.float32),
        mesh=plsc.VectorSubcoreMesh(core_axis_name="c", subcore_axis_name="s"),
        scratch_types=[
            pltpu.VMEM((NB2, BW), jnp.int32),
            pltpu.VMEM((NB2, BW), jnp.int32),
            pltpu.VMEM((BW, C), jnp.float32),
            pltpu.VMEM((64, C), jnp.float32),
            pltpu.VMEM_SHARED((ACC_R, C), jnp.float32),
        ],
    )


def _agg2(h2p, src_t2, dst_t2):
    return _agg2_build()(h2p, src_t2, dst_t2)


_RB = 2000


def _mm1_body(x_ref, w_ref, p_ref, h_ref, dinv_ref):
    dinv = lax.rsqrt(p_ref[...])
    h = jnp.dot(x_ref[...], w_ref[...], preferred_element_type=jnp.float32)
    h_ref[0] = h * dinv
    dinv_ref[...] = dinv


def _mm1(x, w1, deg):
    return pl.pallas_call(
        _mm1_body,
        grid=(N // _RB, H // 128),
        in_specs=[
            pl.BlockSpec((_RB, F_IN), lambda i, j: (i, 0)),
            pl.BlockSpec((F_IN, 128), lambda i, j: (0, j)),
            pl.BlockSpec((_RB, 1), lambda i, j: (i, 0)),
        ],
        out_specs=[
            pl.BlockSpec((1, _RB, 128), lambda i, j: (j, i, 0)),
            pl.BlockSpec((_RB, 1), lambda i, j: (i, 0)),
        ],
        out_shape=[
            jax.ShapeDtypeStruct((H // 128, N, 128), jnp.float32),
            jax.ShapeDtypeStruct((N, 1), jnp.float32),
        ],
    )(x, w1, deg)


def _mm2_body(s1_ref, h1_ref, dinv_ref, b1_ref, w2_ref, out_ref, acc_ref):
    k = pl.program_id(1)
    dinv = dinv_ref[...]
    u = jnp.maximum(dinv * (s1_ref[0] + h1_ref[0]) + b1_ref[...], 0.0)
    part = jnp.dot(u, w2_ref[...], preferred_element_type=jnp.float32)

    @pl.when(k == 0)
    def _():
        acc_ref[...] = part

    @pl.when(k > 0)
    def _():
        acc_ref[...] += part

    @pl.when(k == H // 128 - 1)
    def _():
        out_ref[...] = dinv * acc_ref[...]


def _mm2(s1, h1p, dinv, b1, w2):
    return pl.pallas_call(
        _mm2_body,
        grid=(N // _RB, H // 128),
        in_specs=[
            pl.BlockSpec((1, _RB, 128), lambda i, k: (k, i, 0)),
            pl.BlockSpec((1, _RB, 128), lambda i, k: (k, i, 0)),
            pl.BlockSpec((_RB, 1), lambda i, k: (i, 0)),
            pl.BlockSpec((1, 128), lambda i, k: (0, k)),
            pl.BlockSpec((128, C), lambda i, k: (k, 0)),
        ],
        out_specs=pl.BlockSpec((_RB, C), lambda i, k: (i, 0)),
        out_shape=jax.ShapeDtypeStruct((N, C), jnp.float32),
        scratch_shapes=[pltpu.VMEM((_RB, C), jnp.float32)],
    )(s1, h1p, dinv, b1, w2)


def _fin_body(s2_ref, h2_ref, dinv_ref, b2_ref, out_ref):
    dinv = dinv_ref[...]
    z = dinv * (s2_ref[0] + s2_ref[1] + h2_ref[...]) + b2_ref[...]
    m = jnp.max(z, axis=1, keepdims=True)
    lse = m + jnp.log(jnp.sum(jnp.exp(z - m), axis=1, keepdims=True))
    out_ref[...] = z - lse


def _fin(s2, h2p, dinv, b2):
    return pl.pallas_call(
        _fin_body,
        grid=(N // _RB,),
        in_specs=[
            pl.BlockSpec((2, _RB, C), lambda i: (0, i, 0)),
            pl.BlockSpec((_RB, C), lambda i: (i, 0)),
            pl.BlockSpec((_RB, 1), lambda i: (i, 0)),
            pl.BlockSpec((1, C), lambda i: (0, 0)),
        ],
        out_specs=pl.BlockSpec((_RB, C), lambda i: (i, 0)),
        out_shape=jax.ShapeDtypeStruct((N, C), jnp.float32),
    )(s2, h2p, dinv, b2)


def kernel(x, edge_index, W1, b1, W2, b2):
    src = edge_index[0].astype(jnp.int32)
    dst = edge_index[1].astype(jnp.int32)
    npad = E_PAD - E
    ar = jnp.arange(npad, dtype=jnp.int32)
    src_p = jnp.concatenate([src, (ar * 37) % N])
    dst_p = jnp.concatenate([dst, N + (ar % 16)])
    dst_t = dst_p.reshape(NT, NB, BW)
    src4 = (src_p[None] + (jnp.arange(4, dtype=jnp.int32) * N)[:, None]
            ).reshape(4, NT, NB, BW)
    src_t2 = src_p.reshape(2, NT, NB2, BW)
    dst_t2 = dst_p.reshape(2, NT, NB2, BW)

    deg = _deg_kernel(dst_t)[:N].reshape(N, 1)
    h1p, dinv = _mm1(x, W1, deg)
    s1 = _agg4(h1p.reshape(4 * N, 128), src4, dst_t)
    h2p = _mm2(s1, h1p, dinv, b1.reshape(1, H), W2)
    s2 = _agg2(h2p, src_t2, dst_t2)
    return _fin(s2, h2p, dinv, b2.reshape(1, C))


# speedup vs baseline: 15.5477x; 1.2601x over previous
import functools

import jax
import jax.numpy as jnp
from jax import lax
from jax.experimental import pallas as pl
from jax.experimental.pallas import tpu as pltpu
from jax.experimental.pallas import tpu_sc as plsc

N = 10000
E = 160000
F_IN, H, C = 256, 512, 128

NT = 16
NB = 80
BW = 128
EPT = NB * BW
E_PAD = NT * EPT

ACC_R = 10240
RPT_Z = 640
RPT_W = 624
DEG_R = 10240

def _deg_body(dst_hbm, out_hbm, dstv, ones_row, ones_init, deg_sh):
    c = lax.axis_index("c")
    s = lax.axis_index("s")

    @pl.when(c == 0)
    def _():
        one16 = jnp.full((16,), 1.0, jnp.float32)

        def fill_row(i, carry):
            ones_row[pl.ds(i * 16, 16)] = one16
            return carry

        lax.fori_loop(0, BW // 16, fill_row, 0)

        def fill_init(i, carry):
            ones_init[pl.ds(i * 16, 16)] = one16
            return carry

        lax.fori_loop(0, 640 // 16, fill_init, 0)

        pltpu.sync_copy(ones_init, deg_sh.at[pl.ds(s * 640, 640)])
        pltpu.sync_copy(dst_hbm.at[s], dstv)
        plsc.subcore_barrier()

        def body(j, carry):
            pltpu.sync_copy(ones_row, deg_sh.at[dstv.at[j]], add=True)
            return carry

        lax.fori_loop(0, NB, body, 0)
        plsc.subcore_barrier()
        pltpu.sync_copy(deg_sh.at[pl.ds(s * 640, 640)],
                        out_hbm.at[pl.ds(s * 640, 640)])


@functools.cache
def _deg_kernel_build():
    return pl.kernel(
        _deg_body,
        out_type=jax.ShapeDtypeStruct((DEG_R,), jnp.float32),
        mesh=plsc.VectorSubcoreMesh(core_axis_name="c", subcore_axis_name="s"),
        scratch_types=[
            pltpu.VMEM((NB, BW), jnp.int32),
            pltpu.VMEM((BW,), jnp.float32),
            pltpu.VMEM((640,), jnp.float32),
            pltpu.VMEM_SHARED((DEG_R,), jnp.float32),
        ],
    )


def _deg_kernel(dst_t):
    return _deg_kernel_build()(dst_t)


def _edge_pipe(h_hbm, srcv, dstv, acc_sh, bufs, gsems, ssems, nb):
    def g_start(j, b):
        pltpu.async_copy(h_hbm.at[srcv.at[j]], bufs[b], gsems[b])

    def g_wait(j, b):
        pltpu.make_async_copy(h_hbm.at[srcv.at[j]], bufs[b], gsems[b]).wait()

    def s_start(j, b):
        pltpu.async_copy(bufs[b], acc_sh.at[dstv.at[j]], ssems[b], add=True)

    def s_wait(j, b):
        pltpu.make_async_copy(bufs[b], acc_sh.at[dstv.at[j]], ssems[b]).wait()

    g_start(0, 0)
    g_wait(0, 0)
    g_start(1, 1)
    s_start(0, 0)

    def step(t, carry):
        for off in (0, 1):
            j = 2 * t + 1 + off
            b = (1 + off) % 2
            g_wait(j, b)
            s_wait(j - 1, 1 - b)
            g_start(j + 1, 1 - b)
            s_start(j, b)
        return carry

    lax.fori_loop(0, (nb - 2) // 2, step, 0)
    g_wait(nb - 1, 1)
    s_wait(nb - 2, 0)
    s_start(nb - 1, 1)
    s_wait(nb - 1, 1)


NW = 2
WNB = NB // NW


def _zero_acc_slice(buf0, acc_sh, base, w):
    z16 = jnp.zeros((16,), jnp.float32)

    def zrow(i, carry):
        for k in range(w // 16):
            buf0[i, pl.ds(k * 16, 16)] = z16
        return carry

    lax.fori_loop(0, BW, zrow, 0)
    for m in range(RPT_Z // BW):
        pltpu.sync_copy(buf0, acc_sh.at[pl.ds(base + m * BW, BW)])


def _writeback(acc_sh, out_view, s):
    wb = s * RPT_W

    @pl.when(s < NT - 1)
    def _():
        pltpu.sync_copy(acc_sh.at[pl.ds(wb, RPT_W)],
                        out_view.at[pl.ds(wb, RPT_W)])

    @pl.when(s == NT - 1)
    def _():
        pltpu.sync_copy(acc_sh.at[pl.ds((NT - 1) * RPT_W, N - (NT - 1) * RPT_W)],
                        out_view.at[pl.ds((NT - 1) * RPT_W, N - (NT - 1) * RPT_W)])


def _make_agg(n_chunks, w):
    cpc = n_chunks // 2

    def body(h_hbm, srcoff_hbm, dst_hbm, out_hbm, srcv, dstv,
             buf0, buf1, gs0, gs1, ss0, ss1, acc_sh):
        bufs = [buf0, buf1]
        gsems = [gs0, gs1]
        ssems = [ss0, ss1]
        c = lax.axis_index("c")
        s = lax.axis_index("s")

        for chunk in range(cpc):
            g = c * cpc + chunk
            _zero_acc_slice(buf0, acc_sh, s * RPT_Z, w)
            plsc.subcore_barrier()
            for wnd in range(NW):
                pltpu.sync_copy(srcoff_hbm.at[g, s, wnd], srcv)
                pltpu.sync_copy(dst_hbm.at[s, wnd], dstv)
                _edge_pipe(h_hbm, srcv, dstv, acc_sh, bufs, gsems, ssems, WNB)
            plsc.subcore_barrier()
            _writeback(acc_sh, out_hbm.at[g], s)
            plsc.subcore_barrier()

    return pl.kernel(
        body,
        out_type=jax.ShapeDtypeStruct((n_chunks, N, w), jnp.float32),
        mesh=plsc.VectorSubcoreMesh(core_axis_name="c", subcore_axis_name="s"),
        scratch_types=(
            [pltpu.VMEM((WNB, BW), jnp.int32),
             pltpu.VMEM((WNB, BW), jnp.int32)]
            + [pltpu.VMEM((BW, w), jnp.float32)] * 2
            + [pltpu.SemaphoreType.DMA] * 4
            + [pltpu.VMEM_SHARED((ACC_R, w), jnp.float32)]
        ),
    )


_make_agg = functools.cache(_make_agg)


def _agg4(hflat, src_c, dst_t):
    return _make_agg(4, 128)(hflat, src_c, dst_t)


NB2 = NB // 2


def _agg2_body(h_hbm, src_hbm, dst_hbm, out_hbm, srcv, dstv,
               buf0, buf1, gs0, gs1, ss0, ss1, acc_sh):
    bufs = [buf0, buf1]
    gsems = [gs0, gs1]
    ssems = [ss0, ss1]
    c = lax.axis_index("c")
    s = lax.axis_index("s")
    _zero_acc_slice(buf0, acc_sh, s * RPT_Z, C)
    pltpu.sync_copy(dst_hbm.at[c, s], dstv)
    pltpu.sync_copy(src_hbm.at[c, s], srcv)
    plsc.subcore_barrier()
    _edge_pipe(h_hbm, srcv, dstv, acc_sh, bufs, gsems, ssems, NB2)
    plsc.subcore_barrier()
    _writeback(acc_sh, out_hbm.at[c], s)


@functools.cache
def _agg2_build():
    return pl.kernel(
        _agg2_body,
        out_type=jax.ShapeDtypeStruct((2, N, C), jnp.float32),
        mesh=plsc.VectorSubcoreMesh(core_axis_name="c", subcore_axis_name="s"),
        scratch_types=(
            [pltpu.VMEM((NB2, BW), jnp.int32),
             pltpu.VMEM((NB2, BW), jnp.int32)]
            + [pltpu.VMEM((BW, C), jnp.float32)] * 2
            + [pltpu.SemaphoreType.DMA] * 4
            + [pltpu.VMEM_SHARED((ACC_R, C), jnp.float32)]
        ),
    )


def _agg2(h2p, src_t2, dst_t2):
    return _agg2_build()(h2p, src_t2, dst_t2)


_RB = 2000


def _mm1_body(x_ref, w_ref, p_ref, h_ref, dinv_ref):
    dinv = lax.rsqrt(p_ref[...])
    h = jnp.dot(x_ref[...], w_ref[...], preferred_element_type=jnp.float32)
    h_ref[0] = h * dinv
    dinv_ref[...] = dinv


def _mm1(x, w1, deg):
    return pl.pallas_call(
        _mm1_body,
        grid=(N // _RB, H // 128),
        in_specs=[
            pl.BlockSpec((_RB, F_IN), lambda i, j: (i, 0)),
            pl.BlockSpec((F_IN, 128), lambda i, j: (0, j)),
            pl.BlockSpec((_RB, 1), lambda i, j: (i, 0)),
        ],
        out_specs=[
            pl.BlockSpec((1, _RB, 128), lambda i, j: (j, i, 0)),
            pl.BlockSpec((_RB, 1), lambda i, j: (i, 0)),
        ],
        out_shape=[
            jax.ShapeDtypeStruct((H // 128, N, 128), jnp.float32),
            jax.ShapeDtypeStruct((N, 1), jnp.float32),
        ],
    )(x, w1, deg)


def _mm2_body(s1_ref, h1_ref, dinv_ref, b1_ref, w2_ref, out_ref, acc_ref):
    k = pl.program_id(1)
    dinv = dinv_ref[...]
    u = jnp.maximum(dinv * (s1_ref[0] + h1_ref[0]) + b1_ref[...], 0.0)
    part = jnp.dot(u, w2_ref[...], preferred_element_type=jnp.float32)

    @pl.when(k == 0)
    def _():
        acc_ref[...] = part

    @pl.when(k > 0)
    def _():
        acc_ref[...] += part

    @pl.when(k == H // 128 - 1)
    def _():
        out_ref[...] = dinv * acc_ref[...]


def _mm2(s1, h1p, dinv, b1, w2):
    return pl.pallas_call(
        _mm2_body,
        grid=(N // _RB, H // 128),
        in_specs=[
            pl.BlockSpec((1, _RB, 128), lambda i, k: (k, i, 0)),
            pl.BlockSpec((1, _RB, 128), lambda i, k: (k, i, 0)),
            pl.BlockSpec((_RB, 1), lambda i, k: (i, 0)),
            pl.BlockSpec((1, 128), lambda i, k: (0, k)),
            pl.BlockSpec((128, C), lambda i, k: (k, 0)),
        ],
        out_specs=pl.BlockSpec((_RB, C), lambda i, k: (i, 0)),
        out_shape=jax.ShapeDtypeStruct((N, C), jnp.float32),
        scratch_shapes=[pltpu.VMEM((_RB, C), jnp.float32)],
    )(s1, h1p, dinv, b1, w2)


def _fin_body(s2_ref, h2_ref, dinv_ref, b2_ref, out_ref):
    dinv = dinv_ref[...]
    z = dinv * (s2_ref[0] + s2_ref[1] + h2_ref[...]) + b2_ref[...]
    m = jnp.max(z, axis=1, keepdims=True)
    lse = m + jnp.log(jnp.sum(jnp.exp(z - m), axis=1, keepdims=True))
    out_ref[...] = z - lse


def _fin(s2, h2p, dinv, b2):
    return pl.pallas_call(
        _fin_body,
        grid=(N // _RB,),
        in_specs=[
            pl.BlockSpec((2, _RB, C), lambda i: (0, i, 0)),
            pl.BlockSpec((_RB, C), lambda i: (i, 0)),
            pl.BlockSpec((_RB, 1), lambda i: (i, 0)),
            pl.BlockSpec((1, C), lambda i: (0, 0)),
        ],
        out_specs=pl.BlockSpec((_RB, C), lambda i: (i, 0)),
        out_shape=jax.ShapeDtypeStruct((N, C), jnp.float32),
    )(s2, h2p, dinv, b2)


def kernel(x, edge_index, W1, b1, W2, b2):
    src = edge_index[0].astype(jnp.int32)
    dst = edge_index[1].astype(jnp.int32)
    npad = E_PAD - E
    ar = jnp.arange(npad, dtype=jnp.int32)
    src_p = jnp.concatenate([src, (ar * 37) % N])
    dst_p = jnp.concatenate([dst, N + (ar % 16)])
    dst_t = dst_p.reshape(NT, NB, BW)
    dst_t1 = dst_p.reshape(NT, NW, WNB, BW)
    src4 = (src_p[None] + (jnp.arange(4, dtype=jnp.int32) * N)[:, None]
            ).reshape(4, NT, NW, WNB, BW)
    src_t2 = src_p.reshape(2, NT, NB2, BW)
    dst_t2 = dst_p.reshape(2, NT, NB2, BW)

    deg = _deg_kernel(dst_t)[:N].reshape(N, 1)
    h1p, dinv = _mm1(x, W1, deg)
    s1 = _agg4(h1p.reshape(4 * N, 128), src4, dst_t1)
    h2p = _mm2(s1, h1p, dinv, b1.reshape(1, H), W2)
    s2 = _agg2(h2p, src_t2, dst_t2)
    return _fin(s2, h2p, dinv, b2.reshape(1, C))


# speedup vs baseline: 18.4838x; 1.1888x over previous
import functools

import jax
import jax.numpy as jnp
from jax import lax
from jax.experimental import pallas as pl
from jax.experimental.pallas import tpu as pltpu
from jax.experimental.pallas import tpu_sc as plsc

N = 10000
E = 160000
F_IN, H, C = 256, 512, 128

NT = 16
EPT = 10240
E_PAD = NT * EPT

BW = 64
ND = 4
NBT = EPT // BW
NW = 4
WNB = NBT // NW

DBW = 128
DNB = EPT // DBW

NB2 = NBT // 2
NW2 = NB2 // WNB

ACC_R = 10240
RPT_Z = 640
RPT_W = 624
DEG_R = 10240


def _deg_body(dst_hbm, out_hbm, dstv, ones_row, ones_init, deg_sh):
    c = lax.axis_index("c")
    s = lax.axis_index("s")

    @pl.when(c == 0)
    def _():
        one16 = jnp.full((16,), 1.0, jnp.float32)

        def fill_row(i, carry):
            ones_row[pl.ds(i * 16, 16)] = one16
            return carry

        lax.fori_loop(0, DBW // 16, fill_row, 0)

        def fill_init(i, carry):
            ones_init[pl.ds(i * 16, 16)] = one16
            return carry

        lax.fori_loop(0, 640 // 16, fill_init, 0)

        pltpu.sync_copy(ones_init, deg_sh.at[pl.ds(s * 640, 640)])
        pltpu.sync_copy(dst_hbm.at[s], dstv)
        plsc.subcore_barrier()

        def body(j, carry):
            pltpu.sync_copy(ones_row, deg_sh.at[dstv.at[j]], add=True)
            return carry

        lax.fori_loop(0, DNB, body, 0)
        plsc.subcore_barrier()
        pltpu.sync_copy(deg_sh.at[pl.ds(s * 640, 640)],
                        out_hbm.at[pl.ds(s * 640, 640)])


@functools.cache
def _deg_kernel_build():
    return pl.kernel(
        _deg_body,
        out_type=jax.ShapeDtypeStruct((DEG_R,), jnp.float32),
        mesh=plsc.VectorSubcoreMesh(core_axis_name="c", subcore_axis_name="s"),
        scratch_types=[
            pltpu.VMEM((DNB, DBW), jnp.int32),
            pltpu.VMEM((DBW,), jnp.float32),
            pltpu.VMEM((640,), jnp.float32),
            pltpu.VMEM_SHARED((DEG_R,), jnp.float32),
        ],
    )


def _deg_kernel(dst_t):
    return _deg_kernel_build()(dst_t)


def _edge_pipe(h_hbm, srcv, dstv, acc_sh, bufs, gsems, ssems, nb):
    def g_start(j, b):
        pltpu.async_copy(h_hbm.at[srcv.at[j]], bufs[b], gsems[b])

    def g_wait(j, b):
        pltpu.make_async_copy(h_hbm.at[srcv.at[j]], bufs[b], gsems[b]).wait()

    def s_start(j, b):
        pltpu.async_copy(bufs[b], acc_sh.at[dstv.at[j]], ssems[b], add=True)

    def s_wait(j, b):
        pltpu.make_async_copy(bufs[b], acc_sh.at[dstv.at[j]], ssems[b]).wait()

    for b in range(ND):
        g_start(b, b)
    g_wait(0, 0)
    s_start(0, 0)
    for j in range(1, ND):
        g_wait(j, j)
        s_wait(j - 1, j - 1)
        g_start(j + ND - 1, j - 1)
        s_start(j, j)

    def step(t, carry):
        for b in range(ND):
            j = ND * t + b
            g_wait(j, b)
            s_wait(j - 1, (b - 1) % ND)
            g_start(j + ND - 1, (b - 1) % ND)
            s_start(j, b)
        return carry

    lax.fori_loop(1, nb // ND - 1, step, 0)
    j0 = nb - ND
    g_wait(j0, 0)
    s_wait(j0 - 1, ND - 1)
    g_start(nb - 1, ND - 1)
    s_start(j0, 0)
    for b in range(1, ND):
        g_wait(j0 + b, b)
        s_start(j0 + b, b)
    for b in range(ND):
        s_wait(j0 + b, b)


def _zero_acc_slice(buf0, acc_sh, base):
    z16 = jnp.zeros((16,), jnp.float32)

    def zrow(i, carry):
        for k in range(128 // 16):
            buf0[i, pl.ds(k * 16, 16)] = z16
        return carry

    lax.fori_loop(0, BW, zrow, 0)
    for m in range(RPT_Z // BW):
        pltpu.sync_copy(buf0, acc_sh.at[pl.ds(base + m * BW, BW)])


def _writeback(acc_sh, out_view, s):
    wb = s * RPT_W

    @pl.when(s < NT - 1)
    def _():
        pltpu.sync_copy(acc_sh.at[pl.ds(wb, RPT_W)],
                        out_view.at[pl.ds(wb, RPT_W)])

    @pl.when(s == NT - 1)
    def _():
        pltpu.sync_copy(acc_sh.at[pl.ds((NT - 1) * RPT_W, N - (NT - 1) * RPT_W)],
                        out_view.at[pl.ds((NT - 1) * RPT_W, N - (NT - 1) * RPT_W)])


def _agg1_body(h_hbm, srcoff_hbm, dst_hbm, out_hbm, srcv, dstv,
               buf0, buf1, buf2, buf3, gs0, gs1, gs2, gs3, ss0, ss1, ss2, ss3,
               acc_sh):
    bufs = [buf0, buf1, buf2, buf3]
    gsems = [gs0, gs1, gs2, gs3]
    ssems = [ss0, ss1, ss2, ss3]
    c = lax.axis_index("c")
    s = lax.axis_index("s")

    for chunk in range(2):
        g = c * 2 + chunk
        _zero_acc_slice(buf0, acc_sh, s * RPT_Z)
        plsc.subcore_barrier()
        for wnd in range(NW):
            pltpu.sync_copy(srcoff_hbm.at[g, s, wnd], srcv)
            pltpu.sync_copy(dst_hbm.at[s, wnd], dstv)
            _edge_pipe(h_hbm, srcv, dstv, acc_sh, bufs, gsems, ssems, WNB)
        plsc.subcore_barrier()
        _writeback(acc_sh, out_hbm.at[g], s)
        plsc.subcore_barrier()


@functools.cache
def _agg1_build():
    return pl.kernel(
        _agg1_body,
        out_type=jax.ShapeDtypeStruct((4, N, 128), jnp.float32),
        mesh=plsc.VectorSubcoreMesh(core_axis_name="c", subcore_axis_name="s"),
        scratch_types=(
            [pltpu.VMEM((WNB, BW), jnp.int32),
             pltpu.VMEM((WNB, BW), jnp.int32)]
            + [pltpu.VMEM((BW, 128), jnp.float32)] * ND
            + [pltpu.SemaphoreType.DMA] * (2 * ND)
            + [pltpu.VMEM_SHARED((ACC_R, 128), jnp.float32)]
        ),
    )


def _agg1(hflat, src_c, dst_t):
    return _agg1_build()(hflat, src_c, dst_t)


def _agg2_body(h_hbm, src_hbm, dst_hbm, out_hbm, srcv, dstv,
               buf0, buf1, buf2, buf3, gs0, gs1, gs2, gs3, ss0, ss1, ss2, ss3,
               acc_sh):
    bufs = [buf0, buf1, buf2, buf3]
    gsems = [gs0, gs1, gs2, gs3]
    ssems = [ss0, ss1, ss2, ss3]
    c = lax.axis_index("c")
    s = lax.axis_index("s")
    _zero_acc_slice(buf0, acc_sh, s * RPT_Z)
    plsc.subcore_barrier()
    for wnd in range(NW2):
        pltpu.sync_copy(src_hbm.at[c, s, wnd], srcv)
        pltpu.sync_copy(dst_hbm.at[c, s, wnd], dstv)
        _edge_pipe(h_hbm, srcv, dstv, acc_sh, bufs, gsems, ssems, WNB)
    plsc.subcore_barrier()
    _writeback(acc_sh, out_hbm.at[c], s)


@functools.cache
def _agg2_build():
    return pl.kernel(
        _agg2_body,
        out_type=jax.ShapeDtypeStruct((2, N, C), jnp.float32),
        mesh=plsc.VectorSubcoreMesh(core_axis_name="c", subcore_axis_name="s"),
        scratch_types=(
            [pltpu.VMEM((WNB, BW), jnp.int32),
             pltpu.VMEM((WNB, BW), jnp.int32)]
            + [pltpu.VMEM((BW, C), jnp.float32)] * ND
            + [pltpu.SemaphoreType.DMA] * (2 * ND)
            + [pltpu.VMEM_SHARED((ACC_R, C), jnp.float32)]
        ),
    )


def _agg2(h2p, src_t2, dst_t2):
    return _agg2_build()(h2p, src_t2, dst_t2)


_RB = 2000


def _mm1_body(x_ref, w_ref, p_ref, h_ref, dinv_ref):
    dinv = lax.rsqrt(p_ref[...])
    h = jnp.dot(x_ref[...], w_ref[...], preferred_element_type=jnp.float32)
    h_ref[0] = h * dinv
    dinv_ref[...] = dinv


def _mm1(x, w1, deg):
    return pl.pallas_call(
        _mm1_body,
        grid=(N // _RB, H // 128),
        in_specs=[
            pl.BlockSpec((_RB, F_IN), lambda i, j: (i, 0)),
            pl.BlockSpec((F_IN, 128), lambda i, j: (0, j)),
            pl.BlockSpec((_RB, 1), lambda i, j: (i, 0)),
        ],
        out_specs=[
            pl.BlockSpec((1, _RB, 128), lambda i, j: (j, i, 0)),
            pl.BlockSpec((_RB, 1), lambda i, j: (i, 0)),
        ],
        out_shape=[
            jax.ShapeDtypeStruct((H // 128, N, 128), jnp.float32),
            jax.ShapeDtypeStruct((N, 1), jnp.float32),
        ],
    )(x, w1, deg)


def _mm2_body(s1_ref, h1_ref, dinv_ref, b1_ref, w2_ref, out_ref, acc_ref):
    k = pl.program_id(1)
    dinv = dinv_ref[...]
    u = jnp.maximum(dinv * (s1_ref[0] + h1_ref[0]) + b1_ref[...], 0.0)
    part = jnp.dot(u, w2_ref[...], preferred_element_type=jnp.float32)

    @pl.when(k == 0)
    def _():
        acc_ref[...] = part

    @pl.when(k > 0)
    def _():
        acc_ref[...] += part

    @pl.when(k == H // 128 - 1)
    def _():
        out_ref[...] = dinv * acc_ref[...]


def _mm2(s1, h1p, dinv, b1, w2):
    return pl.pallas_call(
        _mm2_body,
        grid=(N // _RB, H // 128),
        in_specs=[
            pl.BlockSpec((1, _RB, 128), lambda i, k: (k, i, 0)),
            pl.BlockSpec((1, _RB, 128), lambda i, k: (k, i, 0)),
            pl.BlockSpec((_RB, 1), lambda i, k: (i, 0)),
            pl.BlockSpec((1, 128), lambda i, k: (0, k)),
            pl.BlockSpec((128, C), lambda i, k: (k, 0)),
        ],
        out_specs=pl.BlockSpec((_RB, C), lambda i, k: (i, 0)),
        out_shape=jax.ShapeDtypeStruct((N, C), jnp.float32),
        scratch_shapes=[pltpu.VMEM((_RB, C), jnp.float32)],
    )(s1, h1p, dinv, b1, w2)


def _fin_body(s2_ref, h2_ref, dinv_ref, b2_ref, out_ref):
    dinv = dinv_ref[...]
    z = dinv * (s2_ref[0] + s2_ref[1] + h2_ref[...]) + b2_ref[...]
    m = jnp.max(z, axis=1, keepdims=True)
    lse = m + jnp.log(jnp.sum(jnp.exp(z - m), axis=1, keepdims=True))
    out_ref[...] = z - lse


def _fin(s2, h2p, dinv, b2):
    return pl.pallas_call(
        _fin_body,
        grid=(N // _RB,),
        in_specs=[
            pl.BlockSpec((2, _RB, C), lambda i: (0, i, 0)),
            pl.BlockSpec((_RB, C), lambda i: (i, 0)),
            pl.BlockSpec((_RB, 1), lambda i: (i, 0)),
            pl.BlockSpec((1, C), lambda i: (0, 0)),
        ],
        out_specs=pl.BlockSpec((_RB, C), lambda i: (i, 0)),
        out_shape=jax.ShapeDtypeStruct((N, C), jnp.float32),
    )(s2, h2p, dinv, b2)


def kernel(x, edge_index, W1, b1, W2, b2):
    src = edge_index[0].astype(jnp.int32)
    dst = edge_index[1].astype(jnp.int32)
    npad = E_PAD - E
    ar = jnp.arange(npad, dtype=jnp.int32)
    src_p = jnp.concatenate([src, (ar * 37) % N])
    dst_p = jnp.concatenate([dst, N + (ar % 16)])
    dst_t = dst_p.reshape(NT, DNB, DBW)
    dst_t1 = dst_p.reshape(NT, NW, WNB, BW)
    src4 = (src_p[None] + (jnp.arange(4, dtype=jnp.int32) * N)[:, None]
            ).reshape(4, NT, NW, WNB, BW)
    src_t2 = src_p.reshape(2, NT, NW2, WNB, BW)
    dst_t2 = dst_p.reshape(2, NT, NW2, WNB, BW)

    deg = _deg_kernel(dst_t)[:N].reshape(N, 1)
    h1p, dinv = _mm1(x, W1, deg)
    s1 = _agg1(h1p.reshape(4 * N, 128), src4, dst_t1)
    h2p = _mm2(s1, h1p, dinv, b1.reshape(1, H), W2)
    s2 = _agg2(h2p, src_t2, dst_t2)
    return _fin(s2, h2p, dinv, b2.reshape(1, C))
